# baseline (speedup 1.0000x reference)
import functools

import jax
import jax.numpy as jnp
from jax import lax
from jax.experimental import pallas as pl
from jax.experimental.pallas import tpu as pltpu
from jax.experimental.pallas import tpu_sc as plsc

_NC = 2
_NS = 16
_L = 16
_CHUNK = 128


def _sc_partials(x, ei_r, ni, c, n_chunks):
    nw = _NC * _NS
    rpt = ni // _NS

    mesh = plsc.VectorSubcoreMesh(core_axis_name="c", subcore_axis_name="s")

    @functools.partial(
        pl.kernel,
        mesh=mesh,
        compiler_params=pltpu.CompilerParams(use_tc_tiling_on_sc=False),
        out_type=(
            jax.ShapeDtypeStruct((_NC, ni, c), jnp.float32),
            jax.ShapeDtypeStruct((_NC, ni // 2048, 2048), jnp.float32),
        ),
        scratch_types=[
            pltpu.VMEM((n_chunks, _CHUNK), jnp.int32),
            pltpu.VMEM((n_chunks, _CHUNK), jnp.int32),
            pltpu.VMEM((8, _CHUNK, c), jnp.float32),
            pltpu.VMEM((_CHUNK,), jnp.float32),
            pltpu.VMEM((_CHUNK, c), jnp.float32),
            pltpu.VMEM((_CHUNK,), jnp.float32),
            pltpu.VMEM_SHARED((ni, c), jnp.float32),
            pltpu.VMEM_SHARED((ni,), jnp.float32),
            pltpu.SemaphoreType.DMA,
            pltpu.SemaphoreType.DMA,
            pltpu.SemaphoreType.DMA,
            pltpu.SemaphoreType.DMA,
            pltpu.SemaphoreType.DMA,
            pltpu.SemaphoreType.DMA,
            pltpu.SemaphoreType.DMA,
            pltpu.SemaphoreType.DMA,
        ],
    )
    def seg_kernel(x_hbm, ei_hbm, s_out, cnt_out,
                   src_v, dst_v, rows_v, ones_v, zr_v, zc_v,
                   acc_sh, cnt_sh, sem0, sem1, sem2, sem3,
                   sem4, sem5, sem6, sem7):
        cid = lax.axis_index("c")
        sid = lax.axis_index("s")
        wid = cid * _NS + sid

        def fill(i, _):
            ones_v[pl.ds(i * _L, _L)] = jnp.full((_L,), 1.0, jnp.float32)
            zc_v[pl.ds(i * _L, _L)] = jnp.zeros((_L,), jnp.float32)
            return 0
        lax.fori_loop(0, _CHUNK // _L, fill, 0)

        def fillz(i, _):
            zr_v[i] = jnp.zeros((_L,), jnp.float32)
            return 0
        lax.fori_loop(0, _CHUNK, fillz, 0)

        def zero_blk(i, _):
            base = sid * rpt + i * _CHUNK
            pltpu.sync_copy(zr_v, acc_sh.at[pl.ds(base, _CHUNK)])
            pltpu.sync_copy(zc_v, cnt_sh.at[pl.ds(base, _CHUNK)])
            return 0
        lax.fori_loop(0, rpt // _CHUNK, zero_blk, 0)
        plsc.subcore_barrier()

        pltpu.sync_copy(ei_hbm.at[0, wid], src_v)
        pltpu.sync_copy(ei_hbm.at[1, wid], dst_v)

        nring = 8
        sems = (sem0, sem1, sem2, sem3, sem4, sem5, sem6, sem7)
        for q in range(nring - 1):
            pltpu.async_copy(x_hbm.at[src_v.at[q]], rows_v.at[q], sems[q])

        def ring(i, _):
            j0 = i * nring
            for q in range(nring):
                j = j0 + q

                @pl.when(j + nring - 1 < n_chunks)
                def _prefetch(j=j, q=q):
                    b = (q + nring - 1) % nring
                    pltpu.async_copy(x_hbm.at[src_v.at[j + nring - 1]],
                                     rows_v.at[b], sems[b])
                pltpu.sync_copy(ones_v, cnt_sh.at[dst_v.at[j]], add=True)
                pltpu.make_async_copy(x_hbm.at[src_v.at[j]], rows_v.at[q],
                                      sems[q]).wait()
                pltpu.sync_copy(rows_v.at[q], acc_sh.at[dst_v.at[j]],
                                add=True)
            return 0
        lax.fori_loop(0, n_chunks // nring, ring, 0)
        plsc.subcore_barrier()

        pltpu.sync_copy(acc_sh.at[pl.ds(sid * rpt, rpt)],
                        s_out.at[cid, pl.ds(sid * rpt, rpt)])
        pltpu.sync_copy(cnt_sh.at[pl.ds(sid * rpt, rpt)],
                        cnt_out.at[cid, sid])

    return seg_kernel(x, ei_r)


def _tca_body(x_ref, sel_ref, umask_ref, rmat_ref, erep_ref, etile_ref,
              wmat_ref, wbt_ref, out_ref):
    f32 = jnp.float32
    dot = functools.partial(jnp.dot, preferred_element_type=f32)
    xb = dot(dot(sel_ref[...], x_ref[...]) * umask_ref[...], rmat_ref[...])
    xt = dot(xb, etile_ref[...])
    xr = dot(xb, erep_ref[...])
    transf = dot(xr * xt, wmat_ref[...]) + dot(xb, wbt_ref[...])
    out_ref[0] = jnp.transpose(transf)


def _tc_body(x_ref, s_ref, cnt_ref, tr_ref,
             sel_ref, umask_ref, rmat_ref,
             erep_ref, etile_ref, gmat_ref, fbt_ref,
             mw_ref, mb_ref, bvec_ref, m1w_ref, m1b_ref, m2w_ref, m2b_ref,
             out_ref):
    f32 = jnp.float32
    g = pl.program_id(0)
    dot = functools.partial(jnp.dot, preferred_element_type=f32)

    def unpack(a8):
        return dot(dot(sel_ref[...], a8) * umask_ref[...], rmat_ref[...])

    xb = unpack(x_ref[...])
    sb = unpack(s_ref[0] + s_ref[1])
    ca = cnt_ref[0, pl.ds(g, 1), :] + cnt_ref[1, pl.ds(g, 1), :]
    rec_row = 1.0 / jnp.maximum(ca, 1.0)
    r_row = (ca > 0.0).astype(f32)
    rec = jnp.transpose(rec_row)
    r = jnp.transpose(r_row)
    u = sb * rec - r * xb

    ur = dot(u, erep_ref[...])
    xt = dot(xb, etile_ref[...])
    aggr = dot(ur * xt, gmat_ref[...]) + r * dot(xb, fbt_ref[...])
    transf = jnp.transpose(tr_ref[0])

    wgt = jnp.sum(aggr * mw_ref[...], axis=1, keepdims=True) + mb_ref[0, 0]
    conv = jax.nn.sigmoid(aggr * wgt + transf + bvec_ref[...])

    inv_i = 1.0 / xb.shape[0]
    s2 = jnp.sum(conv, axis=0, keepdims=True) * inv_i
    s1x = jnp.sum(xb, axis=0, keepdims=True) * inv_i
    s1 = jnp.concatenate(
        [s1x, jnp.zeros((1, conv.shape[1] - xb.shape[1]), f32)], axis=1)
    z1 = dot(s1, m1w_ref[...]) + m1b_ref[...]
    z2 = dot(s2, m2w_ref[...]) + m2b_ref[...]
    m = jnp.maximum(z1, z2)
    e1 = jnp.exp(z1 - m)
    e2 = jnp.exp(z2 - m)
    g1 = e1 / (e1 + e2)
    g2 = e2 / (e1 + e2)

    cpd, ii = conv.shape[1], xb.shape[0]
    fpad_t = jnp.concatenate(
        [jnp.transpose(xb), jnp.zeros((cpd - xb.shape[1], ii), f32)], axis=0)
    out_ref[0] = (jnp.transpose(g1) * fpad_t
                  + jnp.transpose(g2) * jnp.transpose(conv))


def _tc_transf(x_lin, n, i, c, weights):
    cp = weights["bvec"].shape[1]
    rpb = i * c // 128

    full = lambda shape: pl.BlockSpec(shape, lambda g: (0, 0))
    return pl.pallas_call(
        _tca_body,
        grid=(n,),
        in_specs=[
            pl.BlockSpec((rpb, 128), lambda g: (g, 0)),
            full((i, rpb)),
            full((i, 128)),
            full((128, c)),
            full((c, c * c)),
            full((c, c * c)),
            full((c * c, cp)),
            full((c, cp)),
        ],
        out_specs=pl.BlockSpec((1, cp, i), lambda g: (g, 0, 0)),
        out_shape=jax.ShapeDtypeStruct((n, cp, i), jnp.float32),
    )(x_lin.reshape(n * rpb, 128),
      weights["sel"], weights["umask"], weights["rmat"],
      weights["erep"], weights["etile"], weights["wmat"], weights["wbt"])


def _tc_dense(x_lin, n, i, c, s_part, cnt_part, transf_t, weights):
    cp = weights["bvec"].shape[1]
    rpb = i * c // 128

    full = lambda shape: pl.BlockSpec(shape, lambda g: (0, 0))
    in_specs = [
        pl.BlockSpec((rpb, 128), lambda g: (g, 0)),
        pl.BlockSpec((_NC, rpb, 128), lambda g: (0, g, 0)),
        pl.BlockSpec((_NC, n, i), lambda g: (0, 0, 0)),
        pl.BlockSpec((1, cp, i), lambda g: (g, 0, 0)),
        full((i, rpb)),
        full((i, 128)),
        full((128, c)),
        full((c, c * c)),
        full((c, c * c)),
        full((c * c, cp)),
        full((c, cp)),
        full((1, cp)),
        full((1, 1)),
        full((1, cp)),
        full((cp, cp)),
        full((1, cp)),
        full((cp, cp)),
        full((1, cp)),
    ]
    out_t = pl.pallas_call(
        _tc_body,
        grid=(n,),
        in_specs=in_specs,
        out_specs=pl.BlockSpec((1, cp, i), lambda g: (g, 0, 0)),
        out_shape=jax.ShapeDtypeStruct((n, cp, i), jnp.float32),
    )(x_lin.reshape(n * rpb, 128), s_part.reshape(_NC, n * rpb, 128),
      cnt_part, transf_t,
      weights["sel"], weights["umask"], weights["rmat"],
      weights["erep"], weights["etile"], weights["gmat"],
      weights["fbt"], weights["mw"], weights["mb"],
      weights["bvec"], weights["m1w"], weights["m1b"], weights["m2w"],
      weights["m2b"])
    return jnp.swapaxes(out_t, 1, 2)


def _prep_weights(c, cp, i, F_w, F_b, W_w, W_b, M_w, M_b, B,
                  mlp1_w, mlp1_b, mlp2_w, mlp2_b):
    eye = jnp.eye(c, dtype=jnp.float32)
    ppr = 128 // c
    rows = jnp.arange(i, dtype=jnp.int32)
    sel = (rows[:, None] // ppr
           == jnp.arange(i // ppr, dtype=jnp.int32)[None, :]
           ).astype(jnp.float32)
    umask = (jnp.arange(128, dtype=jnp.int32)[None, :] // c
             == (rows % ppr)[:, None]).astype(jnp.float32)
    rmat = (jnp.arange(128, dtype=jnp.int32)[:, None] % c
            == jnp.arange(c, dtype=jnp.int32)[None, :]).astype(jnp.float32)
    erep = (jnp.arange(c, dtype=jnp.int32)[:, None]
            == (jnp.arange(c * c, dtype=jnp.int32)[None, :] // c)
            ).astype(jnp.float32)
    etile = jnp.tile(eye, (1, c))
    gmat = F_w.reshape(c, cp, c).transpose(0, 2, 1).reshape(c * c, cp)
    wmat = W_w.reshape(c, cp, c).transpose(0, 2, 1).reshape(c * c, cp)
    return {
        "sel": sel,
        "umask": umask,
        "rmat": rmat,
        "erep": erep,
        "etile": etile,
        "gmat": gmat,
        "wmat": wmat,
        "fbt": F_b.reshape(cp, c).T,
        "wbt": W_b.reshape(cp, c).T,
        "mw": M_w.reshape(1, cp),
        "mb": M_b.reshape(1, 1),
        "bvec": B.reshape(1, cp),
        "m1w": mlp1_w,
        "m1b": mlp1_b.reshape(1, cp),
        "m2w": mlp2_w,
        "m2b": mlp2_b.reshape(1, cp),
    }


def kernel(feature_matrix_batch, edge_index, F_w, F_b, W_w, W_b, M_w, M_b,
           B, mlp1_w, mlp1_b, mlp2_w, mlp2_b):
    n, i, c = feature_matrix_batch.shape
    cp = B.shape[0]
    ni = n * i
    e = edge_index.shape[1]
    nw = _NC * _NS
    n_chunks = e // (nw * _CHUNK)

    x = feature_matrix_batch.reshape(ni, c)
    ei_r = edge_index.reshape(2, nw, n_chunks, _CHUNK)

    s_part, cnt_part = _sc_partials(x, ei_r, ni, c, n_chunks)
    weights = _prep_weights(c, cp, i, F_w, F_b, W_w, W_b, M_w, M_b, B,
                            mlp1_w, mlp1_b, mlp2_w, mlp2_b)
    transf_t = _tc_transf(x, n, i, c, weights)
    return _tc_dense(x, n, i, c, s_part, cnt_part, transf_t, weights)


# speedup vs baseline: 1.1966x; 1.1966x over previous
import functools

import jax
import jax.numpy as jnp
from jax import lax
from jax.experimental import pallas as pl
from jax.experimental.pallas import tpu as pltpu
from jax.experimental.pallas import tpu_sc as plsc

_NC = 2
_NS = 16
_L = 16
_CHUNK = 128


def _sc_partials(x, ei_r, ni, c, n_chunks):
    nw = _NC * _NS
    rpt = ni // _NS

    mesh = plsc.VectorSubcoreMesh(core_axis_name="c", subcore_axis_name="s")

    @functools.partial(
        pl.kernel,
        mesh=mesh,
        compiler_params=pltpu.CompilerParams(use_tc_tiling_on_sc=False),
        out_type=(
            jax.ShapeDtypeStruct((_NC, ni, c), jnp.float32),
            jax.ShapeDtypeStruct((_NC, ni // 2048, 2048), jnp.float32),
        ),
        scratch_types=[
            pltpu.VMEM((n_chunks, _CHUNK), jnp.int32),
            pltpu.VMEM((n_chunks, _CHUNK), jnp.int32),
            pltpu.VMEM((8, _CHUNK, c), jnp.float32),
            pltpu.VMEM((_CHUNK,), jnp.float32),
            pltpu.VMEM((_CHUNK, c), jnp.float32),
            pltpu.VMEM((_CHUNK,), jnp.float32),
            pltpu.VMEM_SHARED((ni, c), jnp.float32),
            pltpu.VMEM_SHARED((ni,), jnp.float32),
            pltpu.SemaphoreType.DMA,
            pltpu.SemaphoreType.DMA,
            pltpu.SemaphoreType.DMA,
            pltpu.SemaphoreType.DMA,
            pltpu.SemaphoreType.DMA,
            pltpu.SemaphoreType.DMA,
            pltpu.SemaphoreType.DMA,
            pltpu.SemaphoreType.DMA,
        ],
    )
    def seg_kernel(x_hbm, ei_hbm, s_out, cnt_out,
                   src_v, dst_v, rows_v, ones_v, zr_v, zc_v,
                   acc_sh, cnt_sh, sem0, sem1, sem2, sem3,
                   sem4, sem5, sem6, sem7):
        cid = lax.axis_index("c")
        sid = lax.axis_index("s")
        wid = cid * _NS + sid

        def fill(i, _):
            ones_v[pl.ds(i * _L, _L)] = jnp.full((_L,), 1.0, jnp.float32)
            zc_v[pl.ds(i * _L, _L)] = jnp.zeros((_L,), jnp.float32)
            return 0
        lax.fori_loop(0, _CHUNK // _L, fill, 0)

        def fillz(i, _):
            zr_v[i] = jnp.zeros((_L,), jnp.float32)
            return 0
        lax.fori_loop(0, _CHUNK, fillz, 0)

        def zero_blk(i, _):
            base = sid * rpt + i * _CHUNK
            pltpu.sync_copy(zr_v, acc_sh.at[pl.ds(base, _CHUNK)])
            pltpu.sync_copy(zc_v, cnt_sh.at[pl.ds(base, _CHUNK)])
            return 0
        lax.fori_loop(0, rpt // _CHUNK, zero_blk, 0)
        plsc.subcore_barrier()

        pltpu.sync_copy(ei_hbm.at[0, wid], src_v)
        pltpu.sync_copy(ei_hbm.at[1, wid], dst_v)

        nring = 8
        sems = (sem0, sem1, sem2, sem3, sem4, sem5, sem6, sem7)
        for q in range(nring - 1):
            pltpu.async_copy(x_hbm.at[src_v.at[q]], rows_v.at[q], sems[q])

        def ring(i, _):
            j0 = i * nring
            for q in range(nring):
                j = j0 + q

                @pl.when(j + nring - 1 < n_chunks)
                def _prefetch(j=j, q=q):
                    b = (q + nring - 1) % nring
                    pltpu.async_copy(x_hbm.at[src_v.at[j + nring - 1]],
                                     rows_v.at[b], sems[b])
                pltpu.sync_copy(ones_v, cnt_sh.at[dst_v.at[j]], add=True)
                pltpu.make_async_copy(x_hbm.at[src_v.at[j]], rows_v.at[q],
                                      sems[q]).wait()
                pltpu.sync_copy(rows_v.at[q], acc_sh.at[dst_v.at[j]],
                                add=True)
            return 0
        lax.fori_loop(0, n_chunks // nring, ring, 0)
        plsc.subcore_barrier()

        pltpu.sync_copy(acc_sh.at[pl.ds(sid * rpt, rpt)],
                        s_out.at[cid, pl.ds(sid * rpt, rpt)])
        pltpu.sync_copy(cnt_sh.at[pl.ds(sid * rpt, rpt)],
                        cnt_out.at[cid, sid])

    return seg_kernel(x, ei_r)


def _tc_body(x_ref, s_ref, cnt_ref,
             sel_ref, umask_ref, rmat_ref,
             erep_ref, etile_ref, gmat_ref, wmat_ref, fbt_ref, wbt_ref,
             mw_ref, mb_ref, bvec_ref, m1w_ref, m1b_ref, m2w_ref, m2b_ref,
             out_ref):
    f32 = jnp.float32
    g = pl.program_id(0)
    dot = functools.partial(jnp.dot, preferred_element_type=f32)

    def unpack(a8):
        return dot(dot(sel_ref[...], a8) * umask_ref[...], rmat_ref[...])

    xb = unpack(x_ref[...])
    sb = unpack(s_ref[0] + s_ref[1])
    ca = cnt_ref[0, pl.ds(g, 1), :] + cnt_ref[1, pl.ds(g, 1), :]
    rec_row = 1.0 / jnp.maximum(ca, 1.0)
    r_row = (ca > 0.0).astype(f32)
    rec = jnp.transpose(rec_row)
    r = jnp.transpose(r_row)
    u = sb * rec - r * xb

    ur = dot(u, erep_ref[...])
    xt = dot(xb, etile_ref[...])
    xr = dot(xb, erep_ref[...])
    aggr = dot(ur * xt, gmat_ref[...]) + r * dot(xb, fbt_ref[...])
    transf = dot(xr * xt, wmat_ref[...]) + dot(xb, wbt_ref[...])

    wgt = jnp.sum(aggr * mw_ref[...], axis=1, keepdims=True) + mb_ref[0, 0]
    conv = jax.nn.sigmoid(aggr * wgt + transf + bvec_ref[...])

    inv_i = 1.0 / xb.shape[0]
    s2 = jnp.sum(conv, axis=0, keepdims=True) * inv_i
    s1x = jnp.sum(xb, axis=0, keepdims=True) * inv_i
    s1 = jnp.concatenate(
        [s1x, jnp.zeros((1, conv.shape[1] - xb.shape[1]), f32)], axis=1)
    z1 = dot(s1, m1w_ref[...]) + m1b_ref[...]
    z2 = dot(s2, m2w_ref[...]) + m2b_ref[...]
    m = jnp.maximum(z1, z2)
    e1 = jnp.exp(z1 - m)
    e2 = jnp.exp(z2 - m)
    g1 = e1 / (e1 + e2)
    g2 = e2 / (e1 + e2)

    cpd, ii = conv.shape[1], xb.shape[0]
    fpad_t = jnp.concatenate(
        [jnp.transpose(xb), jnp.zeros((cpd - xb.shape[1], ii), f32)], axis=0)
    out_ref[0] = (jnp.transpose(g1) * fpad_t
                  + jnp.transpose(g2) * jnp.transpose(conv))


def _tc_dense(x_lin, n, i, c, s_part, cnt_part, weights):
    cp = weights["bvec"].shape[1]
    rpb = i * c // 128

    full = lambda shape: pl.BlockSpec(shape, lambda g: (0, 0))
    in_specs = [
        pl.BlockSpec((rpb, 128), lambda g: (g, 0)),
        pl.BlockSpec((_NC, rpb, 128), lambda g: (0, g, 0)),
        pl.BlockSpec((_NC, n, i), lambda g: (0, 0, 0)),
        full((i, rpb)),
        full((i, 128)),
        full((128, c)),
        full((c, c * c)),
        full((c, c * c)),
        full((c * c, cp)),
        full((c * c, cp)),
        full((c, cp)),
        full((c, cp)),
        full((1, cp)),
        full((1, 1)),
        full((1, cp)),
        full((cp, cp)),
        full((1, cp)),
        full((cp, cp)),
        full((1, cp)),
    ]
    out_t = pl.pallas_call(
        _tc_body,
        grid=(n,),
        in_specs=in_specs,
        out_specs=pl.BlockSpec((1, cp, i), lambda g: (g, 0, 0)),
        out_shape=jax.ShapeDtypeStruct((n, cp, i), jnp.float32),
    )(x_lin.reshape(n * rpb, 128), s_part.reshape(_NC, n * rpb, 128),
      cnt_part,
      weights["sel"], weights["umask"], weights["rmat"],
      weights["erep"], weights["etile"], weights["gmat"], weights["wmat"],
      weights["fbt"], weights["wbt"], weights["mw"], weights["mb"],
      weights["bvec"], weights["m1w"], weights["m1b"], weights["m2w"],
      weights["m2b"])
    return jnp.swapaxes(out_t, 1, 2)


def _prep_weights(c, cp, i, F_w, F_b, W_w, W_b, M_w, M_b, B,
                  mlp1_w, mlp1_b, mlp2_w, mlp2_b):
    eye = jnp.eye(c, dtype=jnp.float32)
    ppr = 128 // c
    rows = jnp.arange(i, dtype=jnp.int32)
    sel = (rows[:, None] // ppr
           == jnp.arange(i // ppr, dtype=jnp.int32)[None, :]
           ).astype(jnp.float32)
    umask = (jnp.arange(128, dtype=jnp.int32)[None, :] // c
             == (rows % ppr)[:, None]).astype(jnp.float32)
    rmat = (jnp.arange(128, dtype=jnp.int32)[:, None] % c
            == jnp.arange(c, dtype=jnp.int32)[None, :]).astype(jnp.float32)
    erep = (jnp.arange(c, dtype=jnp.int32)[:, None]
            == (jnp.arange(c * c, dtype=jnp.int32)[None, :] // c)
            ).astype(jnp.float32)
    etile = jnp.tile(eye, (1, c))
    gmat = F_w.reshape(c, cp, c).transpose(0, 2, 1).reshape(c * c, cp)
    wmat = W_w.reshape(c, cp, c).transpose(0, 2, 1).reshape(c * c, cp)
    return {
        "sel": sel,
        "umask": umask,
        "rmat": rmat,
        "erep": erep,
        "etile": etile,
        "gmat": gmat,
        "wmat": wmat,
        "fbt": F_b.reshape(cp, c).T,
        "wbt": W_b.reshape(cp, c).T,
        "mw": M_w.reshape(1, cp),
        "mb": M_b.reshape(1, 1),
        "bvec": B.reshape(1, cp),
        "m1w": mlp1_w,
        "m1b": mlp1_b.reshape(1, cp),
        "m2w": mlp2_w,
        "m2b": mlp2_b.reshape(1, cp),
    }


def kernel(feature_matrix_batch, edge_index, F_w, F_b, W_w, W_b, M_w, M_b,
           B, mlp1_w, mlp1_b, mlp2_w, mlp2_b):
    n, i, c = feature_matrix_batch.shape
    cp = B.shape[0]
    ni = n * i
    e = edge_index.shape[1]
    nw = _NC * _NS
    n_chunks = e // (nw * _CHUNK)

    x = feature_matrix_batch.reshape(ni, c)
    ei_r = edge_index.reshape(2, nw, n_chunks, _CHUNK)

    s_part, cnt_part = _sc_partials(x, ei_r, ni, c, n_chunks)
    weights = _prep_weights(c, cp, i, F_w, F_b, W_w, W_b, M_w, M_b, B,
                            mlp1_w, mlp1_b, mlp2_w, mlp2_b)
    return _tc_dense(x, n, i, c, s_part, cnt_part, weights)


# speedup vs baseline: 1.2354x; 1.0324x over previous
import functools

import jax
import jax.numpy as jnp
from jax import lax
from jax.experimental import pallas as pl
from jax.experimental.pallas import tpu as pltpu
from jax.experimental.pallas import tpu_sc as plsc

_NC = 2
_NS = 16
_L = 16
_CHUNK = 128


def _sc_partials(x, ei_r, ni, c, n_chunks):
    nw = _NC * _NS
    rpt = ni // _NS

    mesh = plsc.VectorSubcoreMesh(core_axis_name="c", subcore_axis_name="s")

    @functools.partial(
        pl.kernel,
        mesh=mesh,
        compiler_params=pltpu.CompilerParams(use_tc_tiling_on_sc=False),
        out_type=(
            jax.ShapeDtypeStruct((_NC, ni, c), jnp.float32),
            jax.ShapeDtypeStruct((_NC, ni // 2048, 2048), jnp.float32),
        ),
        scratch_types=[
            pltpu.VMEM((n_chunks, _CHUNK), jnp.int32),
            pltpu.VMEM((n_chunks, _CHUNK), jnp.int32),
            pltpu.VMEM((8, _CHUNK, c), jnp.float32),
            pltpu.VMEM((_CHUNK,), jnp.float32),
            pltpu.VMEM((_CHUNK, c), jnp.float32),
            pltpu.VMEM((_CHUNK,), jnp.float32),
            pltpu.VMEM_SHARED((ni, c), jnp.float32),
            pltpu.VMEM_SHARED((ni,), jnp.float32),
            pltpu.SemaphoreType.DMA,
            pltpu.SemaphoreType.DMA,
            pltpu.SemaphoreType.DMA,
            pltpu.SemaphoreType.DMA,
            pltpu.SemaphoreType.DMA,
            pltpu.SemaphoreType.DMA,
            pltpu.SemaphoreType.DMA,
            pltpu.SemaphoreType.DMA,
        ],
    )
    def seg_kernel(x_hbm, ei_hbm, s_out, cnt_out,
                   src_v, dst_v, rows_v, ones_v, zr_v, zc_v,
                   acc_sh, cnt_sh, sem0, sem1, sem2, sem3,
                   sem4, sem5, sem6, sem7):
        cid = lax.axis_index("c")
        sid = lax.axis_index("s")
        wid = cid * _NS + sid

        def fill(i, _):
            ones_v[pl.ds(i * _L, _L)] = jnp.full((_L,), 1.0, jnp.float32)
            zc_v[pl.ds(i * _L, _L)] = jnp.zeros((_L,), jnp.float32)
            return 0
        lax.fori_loop(0, _CHUNK // _L, fill, 0)

        def fillz(i, _):
            zr_v[i] = jnp.zeros((_L,), jnp.float32)
            return 0
        lax.fori_loop(0, _CHUNK, fillz, 0)

        def zero_blk(i, _):
            base = sid * rpt + i * _CHUNK
            pltpu.sync_copy(zr_v, acc_sh.at[pl.ds(base, _CHUNK)])
            pltpu.sync_copy(zc_v, cnt_sh.at[pl.ds(base, _CHUNK)])
            return 0
        lax.fori_loop(0, rpt // _CHUNK, zero_blk, 0)
        plsc.subcore_barrier()

        pltpu.sync_copy(ei_hbm.at[0, wid], src_v)
        pltpu.sync_copy(ei_hbm.at[1, wid], dst_v)

        nring = 8
        sems = (sem0, sem1, sem2, sem3, sem4, sem5, sem6, sem7)
        for q in range(nring - 1):
            pltpu.async_copy(x_hbm.at[src_v.at[q]], rows_v.at[q], sems[q])

        def ring(i, _):
            j0 = i * nring
            for q in range(nring):
                j = j0 + q

                @pl.when(j + nring - 1 < n_chunks)
                def _prefetch(j=j, q=q):
                    b = (q + nring - 1) % nring
                    pltpu.async_copy(x_hbm.at[src_v.at[j + nring - 1]],
                                     rows_v.at[b], sems[b])
                pltpu.sync_copy(ones_v, cnt_sh.at[dst_v.at[j]], add=True)
                pltpu.make_async_copy(x_hbm.at[src_v.at[j]], rows_v.at[q],
                                      sems[q]).wait()
                pltpu.sync_copy(rows_v.at[q], acc_sh.at[dst_v.at[j]],
                                add=True)
            return 0
        lax.fori_loop(0, n_chunks // nring, ring, 0)
        plsc.subcore_barrier()

        pltpu.sync_copy(acc_sh.at[pl.ds(sid * rpt, rpt)],
                        s_out.at[cid, pl.ds(sid * rpt, rpt)])
        pltpu.sync_copy(cnt_sh.at[pl.ds(sid * rpt, rpt)],
                        cnt_out.at[cid, sid])

    return seg_kernel(x, ei_r)


def _tc_body(x_ref, s_ref, cnt_ref,
             sel_ref, umask_ref, rmat_ref,
             erep_ref, etile_ref, gmat_ref, wmat_ref, fbt_ref, wbt_ref,
             mw_ref, mb_ref, bvec_ref, m1w_ref, m1b_ref, m2w_ref, m2b_ref,
             out_ref):
    f32 = jnp.float32
    g = pl.program_id(0)
    dot = functools.partial(jnp.dot, preferred_element_type=f32)

    both = dot(sel_ref[...],
               jnp.concatenate([x_ref[...], s_ref[0] + s_ref[1]], axis=1))
    yx = both[:, :128] * umask_ref[...]
    ys = both[:, 128:] * umask_ref[...]
    ca = cnt_ref[0, pl.ds(g, 1), :] + cnt_ref[1, pl.ds(g, 1), :]
    rec_row = 1.0 / jnp.maximum(ca, 1.0)
    r_row = (ca > 0.0).astype(f32)
    rec = jnp.transpose(rec_row)
    r = jnp.transpose(r_row)
    w = ys * rec - yx * r

    xb = dot(yx, rmat_ref[...])
    ur = dot(w, erep_ref[...])
    xtxr = dot(yx, etile_ref[...])
    xt = xtxr[:, :256]
    xr = xtxr[:, 256:]
    aggr = dot(ur * xt, gmat_ref[...]) + r * dot(xb, fbt_ref[...])
    transf = dot(xr * xt, wmat_ref[...]) + dot(xb, wbt_ref[...])

    wgt = jnp.sum(aggr * mw_ref[...], axis=1, keepdims=True) + mb_ref[0, 0]
    conv = jax.nn.sigmoid(aggr * wgt + transf + bvec_ref[...])

    inv_i = 1.0 / xb.shape[0]
    s2 = jnp.sum(conv, axis=0, keepdims=True) * inv_i
    s1x = jnp.sum(xb, axis=0, keepdims=True) * inv_i
    s1 = jnp.concatenate(
        [s1x, jnp.zeros((1, conv.shape[1] - xb.shape[1]), f32)], axis=1)
    z1 = dot(s1, m1w_ref[...]) + m1b_ref[...]
    z2 = dot(s2, m2w_ref[...]) + m2b_ref[...]
    m = jnp.maximum(z1, z2)
    e1 = jnp.exp(z1 - m)
    e2 = jnp.exp(z2 - m)
    g1 = e1 / (e1 + e2)
    g2 = e2 / (e1 + e2)

    cpd, ii = conv.shape[1], xb.shape[0]
    fpad_t = jnp.concatenate(
        [jnp.transpose(xb), jnp.zeros((cpd - xb.shape[1], ii), f32)], axis=0)
    out_ref[0] = (jnp.transpose(g1) * fpad_t
                  + jnp.transpose(g2) * jnp.transpose(conv))


def _tc_dense(x_lin, n, i, c, s_part, cnt_part, weights):
    cp = weights["bvec"].shape[1]
    rpb = i * c // 128

    full = lambda shape: pl.BlockSpec(shape, lambda g: (0, 0))
    in_specs = [
        pl.BlockSpec((rpb, 128), lambda g: (g, 0)),
        pl.BlockSpec((_NC, rpb, 128), lambda g: (0, g, 0)),
        pl.BlockSpec((_NC, n, i), lambda g: (0, 0, 0)),
        full((i, rpb)),
        full((i, 128)),
        full((128, c)),
        full((128, c * c)),
        full((128, 2 * c * c)),
        full((c * c, cp)),
        full((c * c, cp)),
        full((c, cp)),
        full((c, cp)),
        full((1, cp)),
        full((1, 1)),
        full((1, cp)),
        full((cp, cp)),
        full((1, cp)),
        full((cp, cp)),
        full((1, cp)),
    ]
    out_t = pl.pallas_call(
        _tc_body,
        grid=(n,),
        in_specs=in_specs,
        out_specs=pl.BlockSpec((1, cp, i), lambda g: (g, 0, 0)),
        out_shape=jax.ShapeDtypeStruct((n, cp, i), jnp.float32),
    )(x_lin.reshape(n * rpb, 128), s_part.reshape(_NC, n * rpb, 128),
      cnt_part,
      weights["sel"], weights["umask"], weights["rmat"],
      weights["erep"], weights["etile"], weights["gmat"], weights["wmat"],
      weights["fbt"], weights["wbt"], weights["mw"], weights["mb"],
      weights["bvec"], weights["m1w"], weights["m1b"], weights["m2w"],
      weights["m2b"])
    return jnp.swapaxes(out_t, 1, 2)


def _prep_weights(c, cp, i, F_w, F_b, W_w, W_b, M_w, M_b, B,
                  mlp1_w, mlp1_b, mlp2_w, mlp2_b):
    eye = jnp.eye(c, dtype=jnp.float32)
    ppr = 128 // c
    rows = jnp.arange(i, dtype=jnp.int32)
    sel = (rows[:, None] // ppr
           == jnp.arange(i // ppr, dtype=jnp.int32)[None, :]
           ).astype(jnp.float32)
    umask = (jnp.arange(128, dtype=jnp.int32)[None, :] // c
             == (rows % ppr)[:, None]).astype(jnp.float32)
    rmat = (jnp.arange(128, dtype=jnp.int32)[:, None] % c
            == jnp.arange(c, dtype=jnp.int32)[None, :]).astype(jnp.float32)
    del eye
    lanes = jnp.arange(128, dtype=jnp.int32)[:, None] % c
    cols = jnp.arange(c * c, dtype=jnp.int32)[None, :]
    erep = (lanes == cols // c).astype(jnp.float32)
    etile = jnp.concatenate(
        [(lanes == cols % c).astype(jnp.float32), erep], axis=1)
    gmat = F_w.reshape(c, cp, c).transpose(0, 2, 1).reshape(c * c, cp)
    wmat = W_w.reshape(c, cp, c).transpose(0, 2, 1).reshape(c * c, cp)
    return {
        "sel": sel,
        "umask": umask,
        "rmat": rmat,
        "erep": erep,
        "etile": etile,
        "gmat": gmat,
        "wmat": wmat,
        "fbt": F_b.reshape(cp, c).T,
        "wbt": W_b.reshape(cp, c).T,
        "mw": M_w.reshape(1, cp),
        "mb": M_b.reshape(1, 1),
        "bvec": B.reshape(1, cp),
        "m1w": mlp1_w,
        "m1b": mlp1_b.reshape(1, cp),
        "m2w": mlp2_w,
        "m2b": mlp2_b.reshape(1, cp),
    }


def kernel(feature_matrix_batch, edge_index, F_w, F_b, W_w, W_b, M_w, M_b,
           B, mlp1_w, mlp1_b, mlp2_w, mlp2_b):
    n, i, c = feature_matrix_batch.shape
    cp = B.shape[0]
    ni = n * i
    e = edge_index.shape[1]
    nw = _NC * _NS
    n_chunks = e // (nw * _CHUNK)

    x = feature_matrix_batch.reshape(ni, c)
    ei_r = edge_index.reshape(2, nw, n_chunks, _CHUNK)

    s_part, cnt_part = _sc_partials(x, ei_r, ni, c, n_chunks)
    weights = _prep_weights(c, cp, i, F_w, F_b, W_w, W_b, M_w, M_b, B,
                            mlp1_w, mlp1_b, mlp2_w, mlp2_b)
    return _tc_dense(x, n, i, c, s_part, cnt_part, weights)


# speedup vs baseline: 1.2884x; 1.0429x over previous
import functools

import jax
import jax.numpy as jnp
from jax import lax
from jax.experimental import pallas as pl
from jax.experimental.pallas import tpu as pltpu
from jax.experimental.pallas import tpu_sc as plsc

_NC = 2
_NS = 16
_L = 16
_CHUNK = 128


def _sc_partials(x, ei_r, ni, c, n_chunks):
    nw = _NC * _NS
    rpt = ni // _NS

    mesh = plsc.VectorSubcoreMesh(core_axis_name="c", subcore_axis_name="s")

    @functools.partial(
        pl.kernel,
        mesh=mesh,
        compiler_params=pltpu.CompilerParams(use_tc_tiling_on_sc=False),
        out_type=(
            jax.ShapeDtypeStruct((_NC, ni, c), jnp.float32),
            jax.ShapeDtypeStruct((_NC, ni // 2048, 2048), jnp.float32),
        ),
        scratch_types=[
            pltpu.VMEM((n_chunks, _CHUNK), jnp.int32),
            pltpu.VMEM((n_chunks, _CHUNK), jnp.int32),
            pltpu.VMEM((8, _CHUNK, c), jnp.float32),
            pltpu.VMEM((_CHUNK,), jnp.float32),
            pltpu.VMEM((_CHUNK, c), jnp.float32),
            pltpu.VMEM((_CHUNK,), jnp.float32),
            pltpu.VMEM_SHARED((ni, c), jnp.float32),
            pltpu.VMEM_SHARED((ni,), jnp.float32),
            [pltpu.SemaphoreType.DMA] * 8,
            [pltpu.SemaphoreType.DMA] * 8,
            pltpu.SemaphoreType.DMA,
        ],
    )
    def seg_kernel(x_hbm, ei_hbm, s_out, cnt_out,
                   src_v, dst_v, rows_v, ones_v, zr_v, zc_v,
                   acc_sh, cnt_sh, gsems, ssems, csem):
        cid = lax.axis_index("c")
        sid = lax.axis_index("s")
        wid = cid * _NS + sid

        def fill(i, _):
            ones_v[pl.ds(i * _L, _L)] = jnp.full((_L,), 1.0, jnp.float32)
            zc_v[pl.ds(i * _L, _L)] = jnp.zeros((_L,), jnp.float32)
            return 0
        lax.fori_loop(0, _CHUNK // _L, fill, 0)

        def fillz(i, _):
            zr_v[i] = jnp.zeros((_L,), jnp.float32)
            return 0
        lax.fori_loop(0, _CHUNK, fillz, 0)

        def zero_blk(i, _):
            base = sid * rpt + i * _CHUNK
            pltpu.sync_copy(zr_v, acc_sh.at[pl.ds(base, _CHUNK)])
            pltpu.sync_copy(zc_v, cnt_sh.at[pl.ds(base, _CHUNK)])
            return 0
        lax.fori_loop(0, rpt // _CHUNK, zero_blk, 0)
        plsc.subcore_barrier()

        pltpu.sync_copy(ei_hbm.at[0, wid], src_v)
        pltpu.sync_copy(ei_hbm.at[1, wid], dst_v)

        nring = 8
        for q in range(nring - 1):
            pltpu.async_copy(x_hbm.at[src_v.at[q]], rows_v.at[q], gsems[q])
        pltpu.async_copy(ones_v, cnt_sh.at[dst_v.at[0]], csem, add=True)

        def ring(i, _):
            j0 = i * nring
            for q in range(nring):
                j = j0 + q
                b = (q + nring - 1) % nring

                pltpu.make_async_copy(x_hbm.at[src_v.at[j]], rows_v.at[q],
                                      gsems[q]).wait()
                pltpu.async_copy(rows_v.at[q], acc_sh.at[dst_v.at[j]],
                                 ssems[q], add=True)

                @pl.when(j + 1 < n_chunks)
                def _cnt(j=j):
                    pltpu.make_async_copy(ones_v, cnt_sh.at[dst_v.at[j]],
                                          csem).wait()
                    pltpu.async_copy(ones_v, cnt_sh.at[dst_v.at[j + 1]],
                                     csem, add=True)

                @pl.when(j + nring - 1 < n_chunks)
                def _prefetch(j=j, b=b):
                    @pl.when(j >= 1)
                    def _reuse():
                        pltpu.make_async_copy(
                            rows_v.at[b], acc_sh.at[dst_v.at[j - 1]],
                            ssems[b]).wait()
                    pltpu.async_copy(x_hbm.at[src_v.at[j + nring - 1]],
                                     rows_v.at[b], gsems[b])
            return 0
        lax.fori_loop(0, n_chunks // nring, ring, 0)

        for q in range(nring):
            pltpu.make_async_copy(rows_v.at[q],
                                  acc_sh.at[dst_v.at[n_chunks - nring + q]],
                                  ssems[q]).wait()
        pltpu.make_async_copy(ones_v, cnt_sh.at[dst_v.at[n_chunks - 1]],
                              csem).wait()
        plsc.subcore_barrier()

        pltpu.sync_copy(acc_sh.at[pl.ds(sid * rpt, rpt)],
                        s_out.at[cid, pl.ds(sid * rpt, rpt)])
        pltpu.sync_copy(cnt_sh.at[pl.ds(sid * rpt, rpt)],
                        cnt_out.at[cid, sid])

    return seg_kernel(x, ei_r)


def _tc_body(x_ref, s_ref, cnt_ref,
             sel_ref, umask_ref, rmat_ref,
             erep_ref, etile_ref, gmat_ref, wmat_ref, fbt_ref, wbt_ref,
             mw_ref, mb_ref, bvec_ref, m1w_ref, m1b_ref, m2w_ref, m2b_ref,
             out_ref):
    f32 = jnp.float32
    g = pl.program_id(0)
    dot = functools.partial(jnp.dot, preferred_element_type=f32)

    both = dot(sel_ref[...],
               jnp.concatenate([x_ref[...], s_ref[0] + s_ref[1]], axis=1))
    yx = both[:, :128] * umask_ref[...]
    ys = both[:, 128:] * umask_ref[...]
    ca = cnt_ref[0, pl.ds(g, 1), :] + cnt_ref[1, pl.ds(g, 1), :]
    rec_row = 1.0 / jnp.maximum(ca, 1.0)
    r_row = (ca > 0.0).astype(f32)
    rec = jnp.transpose(rec_row)
    r = jnp.transpose(r_row)
    w = ys * rec - yx * r

    xb = dot(yx, rmat_ref[...])
    ur = dot(w, erep_ref[...])
    xtxr = dot(yx, etile_ref[...])
    xt = xtxr[:, :256]
    xr = xtxr[:, 256:]
    aggr = dot(ur * xt, gmat_ref[...]) + r * dot(xb, fbt_ref[...])
    transf = dot(xr * xt, wmat_ref[...]) + dot(xb, wbt_ref[...])

    wgt = jnp.sum(aggr * mw_ref[...], axis=1, keepdims=True) + mb_ref[0, 0]
    conv = jax.nn.sigmoid(aggr * wgt + transf + bvec_ref[...])

    inv_i = 1.0 / xb.shape[0]
    s2 = jnp.sum(conv, axis=0, keepdims=True) * inv_i
    s1x = jnp.sum(xb, axis=0, keepdims=True) * inv_i
    s1 = jnp.concatenate(
        [s1x, jnp.zeros((1, conv.shape[1] - xb.shape[1]), f32)], axis=1)
    z1 = dot(s1, m1w_ref[...]) + m1b_ref[...]
    z2 = dot(s2, m2w_ref[...]) + m2b_ref[...]
    m = jnp.maximum(z1, z2)
    e1 = jnp.exp(z1 - m)
    e2 = jnp.exp(z2 - m)
    g1 = e1 / (e1 + e2)
    g2 = e2 / (e1 + e2)

    cpd, ii = conv.shape[1], xb.shape[0]
    fpad_t = jnp.concatenate(
        [jnp.transpose(xb), jnp.zeros((cpd - xb.shape[1], ii), f32)], axis=0)
    out_ref[0] = (jnp.transpose(g1) * fpad_t
                  + jnp.transpose(g2) * jnp.transpose(conv))


def _tc_dense(x_lin, n, i, c, s_part, cnt_part, weights):
    cp = weights["bvec"].shape[1]
    rpb = i * c // 128

    full = lambda shape: pl.BlockSpec(shape, lambda g: (0, 0))
    in_specs = [
        pl.BlockSpec((rpb, 128), lambda g: (g, 0)),
        pl.BlockSpec((_NC, rpb, 128), lambda g: (0, g, 0)),
        pl.BlockSpec((_NC, n, i), lambda g: (0, 0, 0)),
        full((i, rpb)),
        full((i, 128)),
        full((128, c)),
        full((128, c * c)),
        full((128, 2 * c * c)),
        full((c * c, cp)),
        full((c * c, cp)),
        full((c, cp)),
        full((c, cp)),
        full((1, cp)),
        full((1, 1)),
        full((1, cp)),
        full((cp, cp)),
        full((1, cp)),
        full((cp, cp)),
        full((1, cp)),
    ]
    out_t = pl.pallas_call(
        _tc_body,
        grid=(n,),
        in_specs=in_specs,
        out_specs=pl.BlockSpec((1, cp, i), lambda g: (g, 0, 0)),
        out_shape=jax.ShapeDtypeStruct((n, cp, i), jnp.float32),
    )(x_lin.reshape(n * rpb, 128), s_part.reshape(_NC, n * rpb, 128),
      cnt_part,
      weights["sel"], weights["umask"], weights["rmat"],
      weights["erep"], weights["etile"], weights["gmat"], weights["wmat"],
      weights["fbt"], weights["wbt"], weights["mw"], weights["mb"],
      weights["bvec"], weights["m1w"], weights["m1b"], weights["m2w"],
      weights["m2b"])
    return jnp.swapaxes(out_t, 1, 2)


def _prep_weights(c, cp, i, F_w, F_b, W_w, W_b, M_w, M_b, B,
                  mlp1_w, mlp1_b, mlp2_w, mlp2_b):
    eye = jnp.eye(c, dtype=jnp.float32)
    ppr = 128 // c
    rows = jnp.arange(i, dtype=jnp.int32)
    sel = (rows[:, None] // ppr
           == jnp.arange(i // ppr, dtype=jnp.int32)[None, :]
           ).astype(jnp.float32)
    umask = (jnp.arange(128, dtype=jnp.int32)[None, :] // c
             == (rows % ppr)[:, None]).astype(jnp.float32)
    rmat = (jnp.arange(128, dtype=jnp.int32)[:, None] % c
            == jnp.arange(c, dtype=jnp.int32)[None, :]).astype(jnp.float32)
    del eye
    lanes = jnp.arange(128, dtype=jnp.int32)[:, None] % c
    cols = jnp.arange(c * c, dtype=jnp.int32)[None, :]
    erep = (lanes == cols // c).astype(jnp.float32)
    etile = jnp.concatenate(
        [(lanes == cols % c).astype(jnp.float32), erep], axis=1)
    gmat = F_w.reshape(c, cp, c).transpose(0, 2, 1).reshape(c * c, cp)
    wmat = W_w.reshape(c, cp, c).transpose(0, 2, 1).reshape(c * c, cp)
    return {
        "sel": sel,
        "umask": umask,
        "rmat": rmat,
        "erep": erep,
        "etile": etile,
        "gmat": gmat,
        "wmat": wmat,
        "fbt": F_b.reshape(cp, c).T,
        "wbt": W_b.reshape(cp, c).T,
        "mw": M_w.reshape(1, cp),
        "mb": M_b.reshape(1, 1),
        "bvec": B.reshape(1, cp),
        "m1w": mlp1_w,
        "m1b": mlp1_b.reshape(1, cp),
        "m2w": mlp2_w,
        "m2b": mlp2_b.reshape(1, cp),
    }


def kernel(feature_matrix_batch, edge_index, F_w, F_b, W_w, W_b, M_w, M_b,
           B, mlp1_w, mlp1_b, mlp2_w, mlp2_b):
    n, i, c = feature_matrix_batch.shape
    cp = B.shape[0]
    ni = n * i
    e = edge_index.shape[1]
    nw = _NC * _NS
    n_chunks = e // (nw * _CHUNK)

    x = feature_matrix_batch.reshape(ni, c)
    ei_r = edge_index.reshape(2, nw, n_chunks, _CHUNK)

    s_part, cnt_part = _sc_partials(x, ei_r, ni, c, n_chunks)
    weights = _prep_weights(c, cp, i, F_w, F_b, W_w, W_b, M_w, M_b, B,
                            mlp1_w, mlp1_b, mlp2_w, mlp2_b)
    return _tc_dense(x, n, i, c, s_part, cnt_part, weights)


# speedup vs baseline: 1.2997x; 1.0087x over previous
import functools

import jax
import jax.numpy as jnp
from jax import lax
from jax.experimental import pallas as pl
from jax.experimental.pallas import tpu as pltpu
from jax.experimental.pallas import tpu_sc as plsc

_NC = 2
_NS = 16
_L = 16
_CHUNK = 128


def _sc_partials(x, ei_r, ni, c, n_chunks):
    nw = _NC * _NS
    rpt = ni // _NS

    mesh = plsc.VectorSubcoreMesh(core_axis_name="c", subcore_axis_name="s")

    @functools.partial(
        pl.kernel,
        mesh=mesh,
        compiler_params=pltpu.CompilerParams(use_tc_tiling_on_sc=False),
        out_type=(
            jax.ShapeDtypeStruct((_NC, ni, c), jnp.float32),
            jax.ShapeDtypeStruct((_NC, ni // 2048, 2048), jnp.float32),
        ),
        scratch_types=[
            pltpu.VMEM((n_chunks, _CHUNK), jnp.int32),
            pltpu.VMEM((n_chunks, _CHUNK), jnp.int32),
            pltpu.VMEM((8, _CHUNK, c), jnp.float32),
            pltpu.VMEM((_CHUNK,), jnp.float32),
            pltpu.VMEM((_CHUNK, c), jnp.float32),
            pltpu.VMEM((_CHUNK,), jnp.float32),
            pltpu.VMEM_SHARED((ni, c), jnp.float32),
            pltpu.VMEM_SHARED((ni,), jnp.float32),
            [pltpu.SemaphoreType.DMA] * 8,
            [pltpu.SemaphoreType.DMA] * 8,
            pltpu.SemaphoreType.DMA,
        ],
    )
    def seg_kernel(x_hbm, ei_hbm, s_out, cnt_out,
                   src_v, dst_v, rows_v, ones_v, zr_v, zc_v,
                   acc_sh, cnt_sh, gsems, ssems, csem):
        cid = lax.axis_index("c")
        sid = lax.axis_index("s")
        wid = cid * _NS + sid

        def fill(i, _):
            ones_v[pl.ds(i * _L, _L)] = jnp.full((_L,), 1.0, jnp.float32)
            zc_v[pl.ds(i * _L, _L)] = jnp.zeros((_L,), jnp.float32)
            return 0
        lax.fori_loop(0, _CHUNK // _L, fill, 0)

        def fillz(i, _):
            zr_v[i] = jnp.zeros((_L,), jnp.float32)
            return 0
        lax.fori_loop(0, _CHUNK, fillz, 0)

        def zero_blk(i, _):
            base = sid * rpt + i * _CHUNK
            pltpu.sync_copy(zr_v, acc_sh.at[pl.ds(base, _CHUNK)])
            pltpu.sync_copy(zc_v, cnt_sh.at[pl.ds(base, _CHUNK)])
            return 0
        lax.fori_loop(0, rpt // _CHUNK, zero_blk, 0)
        plsc.subcore_barrier()

        pltpu.sync_copy(ei_hbm.at[0, wid], src_v)
        pltpu.sync_copy(ei_hbm.at[1, wid], dst_v)

        nring = 8
        for q in range(nring - 1):
            pltpu.async_copy(x_hbm.at[src_v.at[q]], rows_v.at[q], gsems[q])
        pltpu.async_copy(ones_v, cnt_sh.at[dst_v.at[0]], csem, add=True)

        def ring(i, _):
            j0 = i * nring
            for q in range(nring):
                j = j0 + q
                b = (q + nring - 1) % nring

                pltpu.make_async_copy(x_hbm.at[src_v.at[j]], rows_v.at[q],
                                      gsems[q]).wait()
                pltpu.async_copy(rows_v.at[q], acc_sh.at[dst_v.at[j]],
                                 ssems[q], add=True)

                @pl.when(j + 1 < n_chunks)
                def _cnt(j=j):
                    pltpu.make_async_copy(ones_v, cnt_sh.at[dst_v.at[j]],
                                          csem).wait()
                    pltpu.async_copy(ones_v, cnt_sh.at[dst_v.at[j + 1]],
                                     csem, add=True)

                @pl.when(j + nring - 1 < n_chunks)
                def _prefetch(j=j, b=b):
                    @pl.when(j >= 1)
                    def _reuse():
                        pltpu.make_async_copy(
                            rows_v.at[b], acc_sh.at[dst_v.at[j - 1]],
                            ssems[b]).wait()
                    pltpu.async_copy(x_hbm.at[src_v.at[j + nring - 1]],
                                     rows_v.at[b], gsems[b])
            return 0
        lax.fori_loop(0, n_chunks // nring, ring, 0)

        for q in range(nring):
            pltpu.make_async_copy(rows_v.at[q],
                                  acc_sh.at[dst_v.at[n_chunks - nring + q]],
                                  ssems[q]).wait()
        pltpu.make_async_copy(ones_v, cnt_sh.at[dst_v.at[n_chunks - 1]],
                              csem).wait()
        plsc.subcore_barrier()

        pltpu.sync_copy(acc_sh.at[pl.ds(sid * rpt, rpt)],
                        s_out.at[cid, pl.ds(sid * rpt, rpt)])
        pltpu.sync_copy(cnt_sh.at[pl.ds(sid * rpt, rpt)],
                        cnt_out.at[cid, sid])

    return seg_kernel(x, ei_r)


def _tc_body(x_ref, s_ref, cnt_ref,
             sel_ref, umask_ref, rmat_ref,
             erep_ref, etile_ref, gmat_ref, wmat_ref, fbt_ref, wbt_ref,
             mw_ref, mb_ref, bvec_ref, m1w_ref, m1b_ref, m2w_ref, m2b_ref,
             out_ref):
    f32 = jnp.float32
    g = pl.program_id(0)
    dot = functools.partial(jnp.dot, preferred_element_type=f32)
    gpb = out_ref.shape[0]
    rpb = sel_ref.shape[1]

    for k in range(gpb):
        rows = pl.ds(k * rpb, rpb)
        both = dot(sel_ref[...],
                   jnp.concatenate([x_ref[rows, :],
                                    s_ref[0, rows, :] + s_ref[1, rows, :]],
                                   axis=1))
        yx = both[:, :128] * umask_ref[...]
        ys = both[:, 128:] * umask_ref[...]
        ca = (cnt_ref[0, pl.ds(g * gpb + k, 1), :]
              + cnt_ref[1, pl.ds(g * gpb + k, 1), :])
        rec_row = 1.0 / jnp.maximum(ca, 1.0)
        r_row = (ca > 0.0).astype(f32)
        rec = jnp.transpose(rec_row)
        r = jnp.transpose(r_row)
        w = ys * rec - yx * r

        xb = dot(yx, rmat_ref[...])
        ur = dot(w, erep_ref[...])
        xtxr = dot(yx, etile_ref[...])
        xt = xtxr[:, :256]
        xr = xtxr[:, 256:]
        aggr = dot(ur * xt, gmat_ref[...]) + r * dot(xb, fbt_ref[...])
        transf = dot(xr * xt, wmat_ref[...]) + dot(xb, wbt_ref[...])

        wgt = jnp.sum(aggr * mw_ref[...], axis=1, keepdims=True) + mb_ref[0, 0]
        conv = jax.nn.sigmoid(aggr * wgt + transf + bvec_ref[...])

        inv_i = 1.0 / conv.shape[0]
        s2 = jnp.sum(conv, axis=0, keepdims=True) * inv_i
        s1x = jnp.sum(xb, axis=0, keepdims=True) * inv_i
        s1 = jnp.concatenate(
            [s1x, jnp.zeros((1, conv.shape[1] - xb.shape[1]), f32)], axis=1)
        z1 = dot(s1, m1w_ref[...]) + m1b_ref[...]
        z2 = dot(s2, m2w_ref[...]) + m2b_ref[...]
        m = jnp.maximum(z1, z2)
        e1 = jnp.exp(z1 - m)
        e2 = jnp.exp(z2 - m)
        g1 = e1 / (e1 + e2)
        g2 = e2 / (e1 + e2)

        cpd, ii = conv.shape[1], conv.shape[0]
        fpad_t = jnp.concatenate(
            [jnp.transpose(xb), jnp.zeros((cpd - xb.shape[1], ii), f32)],
            axis=0)
        out_ref[k] = (jnp.transpose(g1) * fpad_t
                      + jnp.transpose(g2) * jnp.transpose(conv))


def _tc_dense(x_lin, n, i, c, s_part, cnt_part, weights):
    cp = weights["bvec"].shape[1]
    rpb = i * c // 128
    gpb = 2

    full = lambda shape: pl.BlockSpec(shape, lambda g: (0, 0))
    in_specs = [
        pl.BlockSpec((gpb * rpb, 128), lambda g: (g, 0)),
        pl.BlockSpec((_NC, gpb * rpb, 128), lambda g: (0, g, 0)),
        pl.BlockSpec((_NC, n, i), lambda g: (0, 0, 0)),
        full((i, rpb)),
        full((i, 128)),
        full((128, c)),
        full((128, c * c)),
        full((128, 2 * c * c)),
        full((c * c, cp)),
        full((c * c, cp)),
        full((c, cp)),
        full((c, cp)),
        full((1, cp)),
        full((1, 1)),
        full((1, cp)),
        full((cp, cp)),
        full((1, cp)),
        full((cp, cp)),
        full((1, cp)),
    ]
    out_t = pl.pallas_call(
        _tc_body,
        grid=(n // gpb,),
        in_specs=in_specs,
        out_specs=pl.BlockSpec((gpb, cp, i), lambda g: (g, 0, 0)),
        out_shape=jax.ShapeDtypeStruct((n, cp, i), jnp.float32),
    )(x_lin.reshape(n * rpb, 128), s_part.reshape(_NC, n * rpb, 128),
      cnt_part,
      weights["sel"], weights["umask"], weights["rmat"],
      weights["erep"], weights["etile"], weights["gmat"], weights["wmat"],
      weights["fbt"], weights["wbt"], weights["mw"], weights["mb"],
      weights["bvec"], weights["m1w"], weights["m1b"], weights["m2w"],
      weights["m2b"])
    return jnp.swapaxes(out_t, 1, 2)


def _prep_weights(c, cp, i, F_w, F_b, W_w, W_b, M_w, M_b, B,
                  mlp1_w, mlp1_b, mlp2_w, mlp2_b):
    eye = jnp.eye(c, dtype=jnp.float32)
    ppr = 128 // c
    rows = jnp.arange(i, dtype=jnp.int32)
    sel = (rows[:, None] // ppr
           == jnp.arange(i // ppr, dtype=jnp.int32)[None, :]
           ).astype(jnp.float32)
    umask = (jnp.arange(128, dtype=jnp.int32)[None, :] // c
             == (rows % ppr)[:, None]).astype(jnp.float32)
    rmat = (jnp.arange(128, dtype=jnp.int32)[:, None] % c
            == jnp.arange(c, dtype=jnp.int32)[None, :]).astype(jnp.float32)
    del eye
    lanes = jnp.arange(128, dtype=jnp.int32)[:, None] % c
    cols = jnp.arange(c * c, dtype=jnp.int32)[None, :]
    erep = (lanes == cols // c).astype(jnp.float32)
    etile = jnp.concatenate(
        [(lanes == cols % c).astype(jnp.float32), erep], axis=1)
    gmat = F_w.reshape(c, cp, c).transpose(0, 2, 1).reshape(c * c, cp)
    wmat = W_w.reshape(c, cp, c).transpose(0, 2, 1).reshape(c * c, cp)
    return {
        "sel": sel,
        "umask": umask,
        "rmat": rmat,
        "erep": erep,
        "etile": etile,
        "gmat": gmat,
        "wmat": wmat,
        "fbt": F_b.reshape(cp, c).T,
        "wbt": W_b.reshape(cp, c).T,
        "mw": M_w.reshape(1, cp),
        "mb": M_b.reshape(1, 1),
        "bvec": B.reshape(1, cp),
        "m1w": mlp1_w,
        "m1b": mlp1_b.reshape(1, cp),
        "m2w": mlp2_w,
        "m2b": mlp2_b.reshape(1, cp),
    }


def kernel(feature_matrix_batch, edge_index, F_w, F_b, W_w, W_b, M_w, M_b,
           B, mlp1_w, mlp1_b, mlp2_w, mlp2_b):
    n, i, c = feature_matrix_batch.shape
    cp = B.shape[0]
    ni = n * i
    e = edge_index.shape[1]
    nw = _NC * _NS
    n_chunks = e // (nw * _CHUNK)

    x = feature_matrix_batch.reshape(ni, c)
    ei_r = edge_index.reshape(2, nw, n_chunks, _CHUNK)

    s_part, cnt_part = _sc_partials(x, ei_r, ni, c, n_chunks)
    weights = _prep_weights(c, cp, i, F_w, F_b, W_w, W_b, M_w, M_b, B,
                            mlp1_w, mlp1_b, mlp2_w, mlp2_b)
    return _tc_dense(x, n, i, c, s_part, cnt_part, weights)


# speedup vs baseline: 1.3367x; 1.0285x over previous
import functools

import jax
import jax.numpy as jnp
from jax import lax
from jax.experimental import pallas as pl
from jax.experimental.pallas import tpu as pltpu
from jax.experimental.pallas import tpu_sc as plsc

_NC = 2
_NS = 16
_L = 16
_CHUNK = 128


def _sc_partials(x, ei_r, ni, c, n_chunks):
    nw = _NC * _NS
    rpt = ni // _NS

    mesh = plsc.VectorSubcoreMesh(core_axis_name="c", subcore_axis_name="s")

    @functools.partial(
        pl.kernel,
        mesh=mesh,
        compiler_params=pltpu.CompilerParams(use_tc_tiling_on_sc=False),
        out_type=(
            jax.ShapeDtypeStruct((_NC, ni, c), jnp.float32),
            jax.ShapeDtypeStruct((_NC, ni // 2048, 2048), jnp.float32),
        ),
        scratch_types=[
            pltpu.VMEM((n_chunks, _CHUNK), jnp.int32),
            pltpu.VMEM((n_chunks, _CHUNK), jnp.int32),
            pltpu.VMEM((8, _CHUNK, c), jnp.float32),
            pltpu.VMEM((_CHUNK,), jnp.float32),
            pltpu.VMEM((_CHUNK, c), jnp.float32),
            pltpu.VMEM((_CHUNK,), jnp.float32),
            pltpu.VMEM_SHARED((ni, c), jnp.float32),
            pltpu.VMEM_SHARED((ni,), jnp.float32),
            [pltpu.SemaphoreType.DMA] * 8,
            [pltpu.SemaphoreType.DMA] * 8,
            pltpu.SemaphoreType.DMA,
        ],
    )
    def seg_kernel(x_hbm, ei_hbm, s_out, cnt_out,
                   src_v, dst_v, rows_v, ones_v, zr_v, zc_v,
                   acc_sh, cnt_sh, gsems, ssems, csem):
        cid = lax.axis_index("c")
        sid = lax.axis_index("s")
        wid = cid * _NS + sid

        def fill(i, _):
            ones_v[pl.ds(i * _L, _L)] = jnp.full((_L,), 1.0, jnp.float32)
            zc_v[pl.ds(i * _L, _L)] = jnp.zeros((_L,), jnp.float32)
            return 0
        lax.fori_loop(0, _CHUNK // _L, fill, 0)

        def fillz(i, _):
            zr_v[i] = jnp.zeros((_L,), jnp.float32)
            return 0
        lax.fori_loop(0, _CHUNK, fillz, 0)

        def zero_blk(i, _):
            base = sid * rpt + i * _CHUNK
            pltpu.sync_copy(zr_v, acc_sh.at[pl.ds(base, _CHUNK)])
            pltpu.sync_copy(zc_v, cnt_sh.at[pl.ds(base, _CHUNK)])
            return 0
        lax.fori_loop(0, rpt // _CHUNK, zero_blk, 0)
        plsc.subcore_barrier()

        pltpu.sync_copy(ei_hbm.at[0, wid], src_v)
        pltpu.sync_copy(ei_hbm.at[1, wid], dst_v)

        nring = 8
        for q in range(nring - 1):
            pltpu.async_copy(x_hbm.at[src_v.at[q]], rows_v.at[q], gsems[q])
        pltpu.async_copy(ones_v, cnt_sh.at[dst_v.at[0]], csem, add=True)

        def ring(i, _):
            j0 = i * nring
            for q in range(nring):
                j = j0 + q
                b = (q + nring - 1) % nring

                pltpu.make_async_copy(x_hbm.at[src_v.at[j]], rows_v.at[q],
                                      gsems[q]).wait()
                pltpu.async_copy(rows_v.at[q], acc_sh.at[dst_v.at[j]],
                                 ssems[q], add=True)

                @pl.when(j + 1 < n_chunks)
                def _cnt(j=j):
                    pltpu.make_async_copy(ones_v, cnt_sh.at[dst_v.at[j]],
                                          csem).wait()
                    pltpu.async_copy(ones_v, cnt_sh.at[dst_v.at[j + 1]],
                                     csem, add=True)

                @pl.when(j + nring - 1 < n_chunks)
                def _prefetch(j=j, b=b):
                    @pl.when(j >= 1)
                    def _reuse():
                        pltpu.make_async_copy(
                            rows_v.at[b], acc_sh.at[dst_v.at[j - 1]],
                            ssems[b]).wait()
                    pltpu.async_copy(x_hbm.at[src_v.at[j + nring - 1]],
                                     rows_v.at[b], gsems[b])
            return 0
        lax.fori_loop(0, n_chunks // nring, ring, 0)

        for q in range(nring):
            pltpu.make_async_copy(rows_v.at[q],
                                  acc_sh.at[dst_v.at[n_chunks - nring + q]],
                                  ssems[q]).wait()
        pltpu.make_async_copy(ones_v, cnt_sh.at[dst_v.at[n_chunks - 1]],
                              csem).wait()
        plsc.subcore_barrier()

        pltpu.sync_copy(acc_sh.at[pl.ds(sid * rpt, rpt)],
                        s_out.at[cid, pl.ds(sid * rpt, rpt)])
        pltpu.sync_copy(cnt_sh.at[pl.ds(sid * rpt, rpt)],
                        cnt_out.at[cid, sid])

    return seg_kernel(x, ei_r)


def _tc_body(x_ref, s_ref, cnt_ref,
             sel_ref, umask_ref, rmat_ref,
             erep_ref, etile_ref, gmat_ref, wmat_ref, fbt_ref, wbt_ref,
             mw_ref, mb_ref, bvec_ref, m1w_ref, m1b_ref, m2w_ref, m2b_ref,
             out_ref):
    f32 = jnp.float32
    g = pl.program_id(0)
    dot = functools.partial(jnp.dot, preferred_element_type=f32)
    gpb = out_ref.shape[0]
    rpb = sel_ref.shape[1]

    for k in range(gpb):
        rows = pl.ds(k * rpb, rpb)
        both = dot(sel_ref[...],
                   jnp.concatenate([x_ref[rows, :],
                                    s_ref[0, rows, :] + s_ref[1, rows, :]],
                                   axis=1))
        yx = both[:, :128] * umask_ref[...]
        ys = both[:, 128:] * umask_ref[...]
        ca = (cnt_ref[0, pl.ds(g * gpb + k, 1), :]
              + cnt_ref[1, pl.ds(g * gpb + k, 1), :])
        rec_row = 1.0 / jnp.maximum(ca, 1.0)
        r_row = (ca > 0.0).astype(f32)
        rec = jnp.transpose(rec_row)
        r = jnp.transpose(r_row)
        w = ys * rec - yx * r

        xb = dot(yx, rmat_ref[...])
        ur = dot(w, erep_ref[...])
        xtxr = dot(yx, etile_ref[...])
        xt = xtxr[:, :256]
        xr = xtxr[:, 256:]
        aggr = dot(ur * xt, gmat_ref[...]) + r * dot(xb, fbt_ref[...])
        transf = dot(xr * xt, wmat_ref[...]) + dot(xb, wbt_ref[...])

        wgt = jnp.sum(aggr * mw_ref[...], axis=1, keepdims=True) + mb_ref[0, 0]
        conv = jax.nn.sigmoid(aggr * wgt + transf + bvec_ref[...])

        inv_i = 1.0 / conv.shape[0]
        s2 = jnp.sum(conv, axis=0, keepdims=True) * inv_i
        s1x = jnp.sum(xb, axis=0, keepdims=True) * inv_i
        s1 = jnp.concatenate(
            [s1x, jnp.zeros((1, conv.shape[1] - xb.shape[1]), f32)], axis=1)
        z1 = dot(s1, m1w_ref[...]) + m1b_ref[...]
        z2 = dot(s2, m2w_ref[...]) + m2b_ref[...]
        m = jnp.maximum(z1, z2)
        e1 = jnp.exp(z1 - m)
        e2 = jnp.exp(z2 - m)
        g1 = e1 / (e1 + e2)
        g2 = e2 / (e1 + e2)

        cpd, ii = conv.shape[1], conv.shape[0]
        fpad_t = jnp.concatenate(
            [jnp.transpose(xb), jnp.zeros((cpd - xb.shape[1], ii), f32)],
            axis=0)
        out_ref[k] = (jnp.transpose(g1) * fpad_t
                      + jnp.transpose(g2) * jnp.transpose(conv))


def _tc_dense(x_lin, n, i, c, s_part, cnt_part, weights):
    cp = weights["bvec"].shape[1]
    rpb = i * c // 128
    gpb = 4

    full = lambda shape: pl.BlockSpec(shape, lambda g: (0, 0))
    in_specs = [
        pl.BlockSpec((gpb * rpb, 128), lambda g: (g, 0)),
        pl.BlockSpec((_NC, gpb * rpb, 128), lambda g: (0, g, 0)),
        pl.BlockSpec((_NC, n, i), lambda g: (0, 0, 0)),
        full((i, rpb)),
        full((i, 128)),
        full((128, c)),
        full((128, c * c)),
        full((128, 2 * c * c)),
        full((c * c, cp)),
        full((c * c, cp)),
        full((c, cp)),
        full((c, cp)),
        full((1, cp)),
        full((1, 1)),
        full((1, cp)),
        full((cp, cp)),
        full((1, cp)),
        full((cp, cp)),
        full((1, cp)),
    ]
    out_t = pl.pallas_call(
        _tc_body,
        grid=(n // gpb,),
        in_specs=in_specs,
        out_specs=pl.BlockSpec((gpb, cp, i), lambda g: (g, 0, 0)),
        out_shape=jax.ShapeDtypeStruct((n, cp, i), jnp.float32),
    )(x_lin.reshape(n * rpb, 128), s_part.reshape(_NC, n * rpb, 128),
      cnt_part,
      weights["sel"], weights["umask"], weights["rmat"],
      weights["erep"], weights["etile"], weights["gmat"], weights["wmat"],
      weights["fbt"], weights["wbt"], weights["mw"], weights["mb"],
      weights["bvec"], weights["m1w"], weights["m1b"], weights["m2w"],
      weights["m2b"])
    return jnp.swapaxes(out_t, 1, 2)


def _prep_weights(c, cp, i, F_w, F_b, W_w, W_b, M_w, M_b, B,
                  mlp1_w, mlp1_b, mlp2_w, mlp2_b):
    eye = jnp.eye(c, dtype=jnp.float32)
    ppr = 128 // c
    rows = jnp.arange(i, dtype=jnp.int32)
    sel = (rows[:, None] // ppr
           == jnp.arange(i // ppr, dtype=jnp.int32)[None, :]
           ).astype(jnp.float32)
    umask = (jnp.arange(128, dtype=jnp.int32)[None, :] // c
             == (rows % ppr)[:, None]).astype(jnp.float32)
    rmat = (jnp.arange(128, dtype=jnp.int32)[:, None] % c
            == jnp.arange(c, dtype=jnp.int32)[None, :]).astype(jnp.float32)
    del eye
    lanes = jnp.arange(128, dtype=jnp.int32)[:, None] % c
    cols = jnp.arange(c * c, dtype=jnp.int32)[None, :]
    erep = (lanes == cols // c).astype(jnp.float32)
    etile = jnp.concatenate(
        [(lanes == cols % c).astype(jnp.float32), erep], axis=1)
    gmat = F_w.reshape(c, cp, c).transpose(0, 2, 1).reshape(c * c, cp)
    wmat = W_w.reshape(c, cp, c).transpose(0, 2, 1).reshape(c * c, cp)
    return {
        "sel": sel,
        "umask": umask,
        "rmat": rmat,
        "erep": erep,
        "etile": etile,
        "gmat": gmat,
        "wmat": wmat,
        "fbt": F_b.reshape(cp, c).T,
        "wbt": W_b.reshape(cp, c).T,
        "mw": M_w.reshape(1, cp),
        "mb": M_b.reshape(1, 1),
        "bvec": B.reshape(1, cp),
        "m1w": mlp1_w,
        "m1b": mlp1_b.reshape(1, cp),
        "m2w": mlp2_w,
        "m2b": mlp2_b.reshape(1, cp),
    }


def kernel(feature_matrix_batch, edge_index, F_w, F_b, W_w, W_b, M_w, M_b,
           B, mlp1_w, mlp1_b, mlp2_w, mlp2_b):
    n, i, c = feature_matrix_batch.shape
    cp = B.shape[0]
    ni = n * i
    e = edge_index.shape[1]
    nw = _NC * _NS
    n_chunks = e // (nw * _CHUNK)

    x = feature_matrix_batch.reshape(ni, c)
    ei_r = edge_index.reshape(2, nw, n_chunks, _CHUNK)

    s_part, cnt_part = _sc_partials(x, ei_r, ni, c, n_chunks)
    weights = _prep_weights(c, cp, i, F_w, F_b, W_w, W_b, M_w, M_b, B,
                            mlp1_w, mlp1_b, mlp2_w, mlp2_b)
    return _tc_dense(x, n, i, c, s_part, cnt_part, weights)


# speedup vs baseline: 1.3369x; 1.0002x over previous
import functools

import jax
import jax.numpy as jnp
from jax import lax
from jax.experimental import pallas as pl
from jax.experimental.pallas import tpu as pltpu
from jax.experimental.pallas import tpu_sc as plsc

_NC = 2
_NS = 16
_L = 16
_CHUNK = 128


def _sc_partials(x, ei_r, ni, c, n_chunks):
    nw = _NC * _NS
    rpt = ni // _NS

    mesh = plsc.VectorSubcoreMesh(core_axis_name="c", subcore_axis_name="s")

    @functools.partial(
        pl.kernel,
        mesh=mesh,
        compiler_params=pltpu.CompilerParams(use_tc_tiling_on_sc=False),
        out_type=(
            jax.ShapeDtypeStruct((_NC, ni, c), jnp.float32),
            jax.ShapeDtypeStruct((_NC, ni // 2048, 2048), jnp.float32),
        ),
        scratch_types=[
            pltpu.VMEM((n_chunks, _CHUNK), jnp.int32),
            pltpu.VMEM((n_chunks, _CHUNK), jnp.int32),
            pltpu.VMEM((8, _CHUNK, c), jnp.float32),
            pltpu.VMEM((_CHUNK,), jnp.float32),
            pltpu.VMEM((_CHUNK, c), jnp.float32),
            pltpu.VMEM((_CHUNK,), jnp.float32),
            pltpu.VMEM_SHARED((ni, c), jnp.float32),
            pltpu.VMEM_SHARED((ni,), jnp.float32),
            [pltpu.SemaphoreType.DMA] * 8,
            [pltpu.SemaphoreType.DMA] * 8,
            pltpu.SemaphoreType.DMA,
        ],
    )
    def seg_kernel(x_hbm, ei_hbm, s_out, cnt_out,
                   src_v, dst_v, rows_v, ones_v, zr_v, zc_v,
                   acc_sh, cnt_sh, gsems, ssems, csem):
        cid = lax.axis_index("c")
        sid = lax.axis_index("s")
        wid = cid * _NS + sid

        def fill(i, _):
            ones_v[pl.ds(i * _L, _L)] = jnp.full((_L,), 1.0, jnp.float32)
            zc_v[pl.ds(i * _L, _L)] = jnp.zeros((_L,), jnp.float32)
            return 0
        lax.fori_loop(0, _CHUNK // _L, fill, 0)

        def fillz(i, _):
            zr_v[i] = jnp.zeros((_L,), jnp.float32)
            return 0
        lax.fori_loop(0, _CHUNK, fillz, 0)

        def zero_blk(i, _):
            base = sid * rpt + i * _CHUNK
            pltpu.sync_copy(zr_v, acc_sh.at[pl.ds(base, _CHUNK)])
            pltpu.sync_copy(zc_v, cnt_sh.at[pl.ds(base, _CHUNK)])
            return 0
        lax.fori_loop(0, rpt // _CHUNK, zero_blk, 0)
        plsc.subcore_barrier()

        pltpu.sync_copy(ei_hbm.at[0, wid], src_v)
        pltpu.sync_copy(ei_hbm.at[1, wid], dst_v)

        nring = 8
        for q in range(nring - 1):
            pltpu.async_copy(x_hbm.at[src_v.at[q]], rows_v.at[q], gsems[q])
        pltpu.async_copy(ones_v, cnt_sh.at[dst_v.at[0]], csem, add=True)

        def ring(i, _):
            j0 = i * nring
            for q in range(nring):
                j = j0 + q
                b = (q + nring - 1) % nring

                pltpu.make_async_copy(x_hbm.at[src_v.at[j]], rows_v.at[q],
                                      gsems[q]).wait()
                pltpu.async_copy(rows_v.at[q], acc_sh.at[dst_v.at[j]],
                                 ssems[q], add=True)

                @pl.when(j + 1 < n_chunks)
                def _cnt(j=j):
                    pltpu.make_async_copy(ones_v, cnt_sh.at[dst_v.at[j]],
                                          csem).wait()
                    pltpu.async_copy(ones_v, cnt_sh.at[dst_v.at[j + 1]],
                                     csem, add=True)

                @pl.when(j + nring - 1 < n_chunks)
                def _prefetch(j=j, b=b):
                    @pl.when(j >= 1)
                    def _reuse():
                        pltpu.make_async_copy(
                            rows_v.at[b], acc_sh.at[dst_v.at[j - 1]],
                            ssems[b]).wait()
                    pltpu.async_copy(x_hbm.at[src_v.at[j + nring - 1]],
                                     rows_v.at[b], gsems[b])
            return 0
        lax.fori_loop(0, n_chunks // nring, ring, 0)

        for q in range(nring):
            pltpu.make_async_copy(rows_v.at[q],
                                  acc_sh.at[dst_v.at[n_chunks - nring + q]],
                                  ssems[q]).wait()
        pltpu.make_async_copy(ones_v, cnt_sh.at[dst_v.at[n_chunks - 1]],
                              csem).wait()
        plsc.subcore_barrier()

        pltpu.sync_copy(acc_sh.at[pl.ds(sid * rpt, rpt)],
                        s_out.at[cid, pl.ds(sid * rpt, rpt)])
        pltpu.sync_copy(cnt_sh.at[pl.ds(sid * rpt, rpt)],
                        cnt_out.at[cid, sid])

    return seg_kernel(x, ei_r)


def _tc_body(x_ref, s_ref, cnt_ref,
             sel_ref, umask_ref, rmat_ref,
             erep_ref, etile_ref, gmat_ref, wmat_ref, fbt_ref, wbt_ref,
             mw_ref, mb_ref, bvec_ref, m1w_ref, m1b_ref, m2w_ref, m2b_ref,
             out_ref):
    f32 = jnp.float32
    g = pl.program_id(0)
    dot = functools.partial(jnp.dot, preferred_element_type=f32)
    gpb = out_ref.shape[0]
    rpb = sel_ref.shape[1]

    for k in range(gpb):
        rows = pl.ds(k * rpb, rpb)
        both = dot(sel_ref[...],
                   jnp.concatenate([x_ref[rows, :],
                                    s_ref[0, rows, :] + s_ref[1, rows, :]],
                                   axis=1))
        yx = both[:, :128] * umask_ref[...]
        ys = both[:, 128:] * umask_ref[...]
        ca = (cnt_ref[0, pl.ds(g * gpb + k, 1), :]
              + cnt_ref[1, pl.ds(g * gpb + k, 1), :])
        rec_row = 1.0 / jnp.maximum(ca, 1.0)
        r_row = (ca > 0.0).astype(f32)
        rec = jnp.transpose(rec_row)
        r = jnp.transpose(r_row)
        w = ys * rec - yx * r

        xb = dot(yx, rmat_ref[...])
        ur = dot(w, erep_ref[...])
        xtxr = dot(yx, etile_ref[...])
        xt = xtxr[:, :256]
        xr = xtxr[:, 256:]
        aggr = dot(ur * xt, gmat_ref[...]) + r * dot(xb, fbt_ref[...])
        transf = dot(xr * xt, wmat_ref[...]) + dot(xb, wbt_ref[...])

        wgt = jnp.sum(aggr * mw_ref[...], axis=1, keepdims=True) + mb_ref[0, 0]
        conv = jax.nn.sigmoid(aggr * wgt + transf + bvec_ref[...])

        inv_i = 1.0 / conv.shape[0]
        s2 = jnp.sum(conv, axis=0, keepdims=True) * inv_i
        s1x = jnp.sum(xb, axis=0, keepdims=True) * inv_i
        s1 = jnp.concatenate(
            [s1x, jnp.zeros((1, conv.shape[1] - xb.shape[1]), f32)], axis=1)
        z1 = dot(s1, m1w_ref[...]) + m1b_ref[...]
        z2 = dot(s2, m2w_ref[...]) + m2b_ref[...]
        m = jnp.maximum(z1, z2)
        e1 = jnp.exp(z1 - m)
        e2 = jnp.exp(z2 - m)
        g1 = e1 / (e1 + e2)
        g2 = e2 / (e1 + e2)

        cpd, ii = conv.shape[1], conv.shape[0]
        fpad_t = jnp.concatenate(
            [jnp.transpose(xb), jnp.zeros((cpd - xb.shape[1], ii), f32)],
            axis=0)
        out_ref[k] = (jnp.transpose(g1) * fpad_t
                      + jnp.transpose(g2) * jnp.transpose(conv))


def _tc_dense(x_lin, n, i, c, s_part, cnt_part, weights):
    cp = weights["bvec"].shape[1]
    rpb = i * c // 128
    gpb = 4

    full = lambda shape: pl.BlockSpec(shape, lambda g: (0, 0))
    in_specs = [
        pl.BlockSpec((gpb * rpb, 128), lambda g: (g, 0)),
        pl.BlockSpec((_NC, gpb * rpb, 128), lambda g: (0, g, 0)),
        pl.BlockSpec((_NC, n, i), lambda g: (0, 0, 0)),
        full((i, rpb)),
        full((i, 128)),
        full((128, c)),
        full((128, c * c)),
        full((128, 2 * c * c)),
        full((c * c, cp)),
        full((c * c, cp)),
        full((c, cp)),
        full((c, cp)),
        full((1, cp)),
        full((1, 1)),
        full((1, cp)),
        full((cp, cp)),
        full((1, cp)),
        full((cp, cp)),
        full((1, cp)),
    ]
    out_t = pl.pallas_call(
        _tc_body,
        grid=(n // gpb,),
        in_specs=in_specs,
        out_specs=pl.BlockSpec((gpb, cp, i), lambda g: (g, 0, 0)),
        out_shape=jax.ShapeDtypeStruct((n, cp, i), jnp.float32),
    )(x_lin.reshape(n * rpb, 128), s_part.reshape(_NC, n * rpb, 128),
      cnt_part,
      weights["sel"], weights["umask"], weights["rmat"],
      weights["erep"], weights["etile"], weights["gmat"], weights["wmat"],
      weights["fbt"], weights["wbt"], weights["mw"], weights["mb"],
      weights["bvec"], weights["m1w"], weights["m1b"], weights["m2w"],
      weights["m2b"])
    return jnp.swapaxes(out_t, 1, 2)


def _prep_weights(c, cp, i, F_w, F_b, W_w, W_b, M_w, M_b, B,
                  mlp1_w, mlp1_b, mlp2_w, mlp2_b):
    ppr = 128 // c
    rows = jnp.arange(i, dtype=jnp.int32)
    sel = (rows[:, None] // ppr
           == jnp.arange(i // ppr, dtype=jnp.int32)[None, :]
           ).astype(jnp.float32)
    umask = (jnp.arange(128, dtype=jnp.int32)[None, :] // c
             == (rows % ppr)[:, None]).astype(jnp.float32)
    rmat = (jnp.arange(128, dtype=jnp.int32)[:, None] % c
            == jnp.arange(c, dtype=jnp.int32)[None, :]).astype(jnp.float32)
    lanes = jnp.arange(128, dtype=jnp.int32)[:, None] % c
    cols = jnp.arange(c * c, dtype=jnp.int32)[None, :]
    erep = (lanes == cols // c).astype(jnp.float32)
    etile = jnp.concatenate(
        [(lanes == cols % c).astype(jnp.float32), erep], axis=1)
    gmat = F_w.reshape(c, cp, c).transpose(0, 2, 1).reshape(c * c, cp)
    wmat = W_w.reshape(c, cp, c).transpose(0, 2, 1).reshape(c * c, cp)
    return {
        "sel": sel,
        "umask": umask,
        "rmat": rmat,
        "erep": erep,
        "etile": etile,
        "gmat": gmat,
        "wmat": wmat,
        "fbt": F_b.reshape(cp, c).T,
        "wbt": W_b.reshape(cp, c).T,
        "mw": M_w.reshape(1, cp),
        "mb": M_b.reshape(1, 1),
        "bvec": B.reshape(1, cp),
        "m1w": mlp1_w,
        "m1b": mlp1_b.reshape(1, cp),
        "m2w": mlp2_w,
        "m2b": mlp2_b.reshape(1, cp),
    }


def kernel(feature_matrix_batch, edge_index, F_w, F_b, W_w, W_b, M_w, M_b,
           B, mlp1_w, mlp1_b, mlp2_w, mlp2_b):
    n, i, c = feature_matrix_batch.shape
    cp = B.shape[0]
    ni = n * i
    e = edge_index.shape[1]
    nw = _NC * _NS
    n_chunks = e // (nw * _CHUNK)

    x = feature_matrix_batch.reshape(ni, c)
    ei_r = edge_index.reshape(2, nw, n_chunks, _CHUNK)

    s_part, cnt_part = _sc_partials(x, ei_r, ni, c, n_chunks)
    weights = _prep_weights(c, cp, i, F_w, F_b, W_w, W_b, M_w, M_b, B,
                            mlp1_w, mlp1_b, mlp2_w, mlp2_b)
    return _tc_dense(x, n, i, c, s_part, cnt_part, weights)
